# baseline (speedup 1.0000x reference)
import jax
import jax.numpy as jnp
from jax.experimental import pallas as pl
from jax.experimental.pallas import tpu as pltpu

_N_E = 1024
_E_DIM = 256
_BETA = 0.25
_B = 16
_POS = 1024
_ROWS = _B * _POS
_PB = 256
_PPB = _POS // _PB
_GRID = _ROWS // _PB


def _vq_kernel(z_ref, e_ref, zq_ref, idx_ref, loss_ref, perp_ref, use_ref,
               counts_ref, sse_ref):
    j = pl.program_id(0)

    @pl.when(j == 0)
    def _init():
        counts_ref[...] = jnp.zeros_like(counts_ref)
        sse_ref[0, 0] = jnp.float32(0.0)

    zb = z_ref[0]
    zt = zb.T
    e = e_ref[...]
    z2 = jnp.sum(zt * zt, axis=1, keepdims=True)
    e2 = jnp.sum(e * e, axis=1)
    prod = jax.lax.dot_general(
        zt, e, (((1,), (1,)), ((), ())),
        preferred_element_type=jnp.float32)
    d = z2 + e2[None, :] - 2.0 * prod

    dmin = jnp.min(d, axis=1, keepdims=True)
    iot = jax.lax.broadcasted_iota(jnp.int32, d.shape, 1)
    midx = jnp.min(jnp.where(d == dmin, iot, _N_E), axis=1,
                   keepdims=True)
    onehot = (iot == midx).astype(jnp.float32)
    zq = jax.lax.dot_general(
        onehot, e, (((1,), (0,)), ((), ())),
        preferred_element_type=jnp.float32,
        precision=jax.lax.Precision.HIGHEST)
    zq_ref[0] = zq.T
    idx_ref[...] = midx.reshape(1, 1, _PB)

    diff = zq - zt
    sse_ref[0, 0] += jnp.sum(diff * diff)
    counts_ref[0, :] += jnp.sum(onehot, axis=0)

    @pl.when(j == _GRID - 1)
    def _final():
        n = jnp.float32(_ROWS)
        counts = counts_ref[...]
        avg = counts / n
        loss_ref[...] = jnp.full(
            (1, 1), (1.0 + _BETA) * sse_ref[0, 0] / (n * _E_DIM), jnp.float32)
        ent = jnp.sum(avg * jnp.log(avg + 1e-10), axis=1, keepdims=True)
        perp_ref[...] = jnp.exp(-ent)
        use_ref[...] = jnp.sum((counts > 0.0).astype(jnp.int32), axis=1,
                               keepdims=True)


def kernel(z, embedding_weight):
    zv = z.reshape(_B, _E_DIM, _POS)
    zq_v, idxs, loss, perp, use = pl.pallas_call(
        _vq_kernel,
        grid=(_GRID,),
        in_specs=[
            pl.BlockSpec((1, _E_DIM, _PB), lambda j: (j // _PPB, 0, j % _PPB)),
            pl.BlockSpec((_N_E, _E_DIM), lambda j: (0, 0)),
        ],
        out_specs=[
            pl.BlockSpec((1, _E_DIM, _PB), lambda j: (j // _PPB, 0, j % _PPB)),
            pl.BlockSpec((1, 1, _PB), lambda j: (j, 0, 0)),
            pl.BlockSpec((1, 1), lambda j: (0, 0)),
            pl.BlockSpec((1, 1), lambda j: (0, 0)),
            pl.BlockSpec((1, 1), lambda j: (0, 0)),
        ],
        out_shape=[
            jax.ShapeDtypeStruct((_B, _E_DIM, _POS), jnp.float32),
            jax.ShapeDtypeStruct((_GRID, 1, _PB), jnp.int32),
            jax.ShapeDtypeStruct((1, 1), jnp.float32),
            jax.ShapeDtypeStruct((1, 1), jnp.float32),
            jax.ShapeDtypeStruct((1, 1), jnp.int32),
        ],
        scratch_shapes=[
            pltpu.VMEM((1, _N_E), jnp.float32),
            pltpu.SMEM((1, 1), jnp.float32),
        ],
    )(zv, embedding_weight)
    z_q_out = zq_v.reshape(_B, _E_DIM, 32, 32)
    return (z_q_out, loss[0, 0], perp[0, 0], use[0, 0],
            idxs.reshape(_ROWS))


# speedup vs baseline: 1.4309x; 1.4309x over previous
import jax
import jax.numpy as jnp
from jax.experimental import pallas as pl
from jax.experimental.pallas import tpu as pltpu

_N_E = 1024
_E_DIM = 256
_BETA = 0.25
_B = 16
_POS = 1024
_ROWS = _B * _POS
_PB = 512
_PPB = _POS // _PB
_GRID = _ROWS // _PB


def _vq_kernel(z_ref, e_ref, eth_ref, etm_ref, etl_ref, zq_ref, idx_ref,
               loss_ref, perp_ref, use_ref, counts_ref, sse_ref):
    j = pl.program_id(0)

    @pl.when(j == 0)
    def _init():
        counts_ref[...] = jnp.zeros_like(counts_ref)
        sse_ref[0, 0] = jnp.float32(0.0)

    zb = z_ref[0]
    zt = zb.T
    e = e_ref[...]
    z2 = jnp.sum(zt * zt, axis=1, keepdims=True)
    e2 = jnp.sum(e * e, axis=1)
    prod = jax.lax.dot_general(
        zt, e, (((1,), (1,)), ((), ())),
        preferred_element_type=jnp.float32)
    d = z2 + e2[None, :] - 2.0 * prod

    dmin = jnp.min(d, axis=1, keepdims=True)
    iot = jax.lax.broadcasted_iota(jnp.int32, d.shape, 1)
    midx = jnp.min(jnp.where(d == dmin, iot, _N_E), axis=1,
                   keepdims=True)
    midx_row = midx.reshape(1, _PB)
    iot0 = jax.lax.broadcasted_iota(jnp.int32, (_N_E, _PB), 0)
    onehot_t = (iot0 == midx_row).astype(jnp.bfloat16)
    dims = (((1,), (0,)), ((), ()))
    zq_t = (
        jax.lax.dot_general(eth_ref[...], onehot_t, dims,
                            preferred_element_type=jnp.float32)
        + jax.lax.dot_general(etm_ref[...], onehot_t, dims,
                              preferred_element_type=jnp.float32)
        + jax.lax.dot_general(etl_ref[...], onehot_t, dims,
                              preferred_element_type=jnp.float32)
    )
    zq_ref[0] = zq_t
    idx_ref[...] = midx_row.reshape(1, 1, _PB)

    sse_ref[0, 0] += jnp.sum(dmin)
    counts_ref[...] += jnp.sum(onehot_t.astype(jnp.float32), axis=1,
                               keepdims=True).reshape(1, _N_E)

    @pl.when(j == _GRID - 1)
    def _final():
        n = jnp.float32(_ROWS)
        counts = counts_ref[...]
        avg = counts / n
        loss_ref[...] = jnp.full(
            (1, 1), (1.0 + _BETA) * sse_ref[0, 0] / (n * _E_DIM), jnp.float32)
        ent = jnp.sum(avg * jnp.log(avg + 1e-10), axis=1, keepdims=True)
        perp_ref[...] = jnp.exp(-ent)
        use_ref[...] = jnp.sum((counts > 0.0).astype(jnp.int32), axis=1,
                               keepdims=True)


def kernel(z, embedding_weight):
    zv = z.reshape(_B, _E_DIM, _POS)
    et = embedding_weight.T
    et_hi = et.astype(jnp.bfloat16)
    r1 = et - et_hi.astype(jnp.float32)
    et_mid = r1.astype(jnp.bfloat16)
    et_lo = (r1 - et_mid.astype(jnp.float32)).astype(jnp.bfloat16)
    zq_v, idxs, loss, perp, use = pl.pallas_call(
        _vq_kernel,
        grid=(_GRID,),
        in_specs=[
            pl.BlockSpec((1, _E_DIM, _PB), lambda j: (j // _PPB, 0, j % _PPB)),
            pl.BlockSpec((_N_E, _E_DIM), lambda j: (0, 0)),
            pl.BlockSpec((_E_DIM, _N_E), lambda j: (0, 0)),
            pl.BlockSpec((_E_DIM, _N_E), lambda j: (0, 0)),
            pl.BlockSpec((_E_DIM, _N_E), lambda j: (0, 0)),
        ],
        out_specs=[
            pl.BlockSpec((1, _E_DIM, _PB), lambda j: (j // _PPB, 0, j % _PPB)),
            pl.BlockSpec((1, 1, _PB), lambda j: (j, 0, 0)),
            pl.BlockSpec((1, 1), lambda j: (0, 0)),
            pl.BlockSpec((1, 1), lambda j: (0, 0)),
            pl.BlockSpec((1, 1), lambda j: (0, 0)),
        ],
        out_shape=[
            jax.ShapeDtypeStruct((_B, _E_DIM, _POS), jnp.float32),
            jax.ShapeDtypeStruct((_GRID, 1, _PB), jnp.int32),
            jax.ShapeDtypeStruct((1, 1), jnp.float32),
            jax.ShapeDtypeStruct((1, 1), jnp.float32),
            jax.ShapeDtypeStruct((1, 1), jnp.int32),
        ],
        scratch_shapes=[
            pltpu.VMEM((1, _N_E), jnp.float32),
            pltpu.SMEM((1, 1), jnp.float32),
        ],
    )(zv, embedding_weight, et_hi, et_mid, et_lo)
    z_q_out = zq_v.reshape(_B, _E_DIM, 32, 32)
    return (z_q_out, loss[0, 0], perp[0, 0], use[0, 0],
            idxs.reshape(_ROWS))


# speedup vs baseline: 1.7635x; 1.2325x over previous
import jax
import jax.numpy as jnp
from jax.experimental import pallas as pl
from jax.experimental.pallas import tpu as pltpu

_N_E = 1024
_E_DIM = 256
_BETA = 0.25
_B = 16
_POS = 1024
_ROWS = _B * _POS
_PB = 1024
_PPB = _POS // _PB
_GRID = _ROWS // _PB


def _vq_kernel(z_ref, e_ref, eth_ref, etm_ref, ones_ref, zq_ref, idx_ref,
               loss_ref, perp_ref, use_ref, counts_ref, sse_ref):
    j = pl.program_id(0)

    @pl.when(j == 0)
    def _init():
        counts_ref[...] = jnp.zeros_like(counts_ref)
        sse_ref[0, 0] = jnp.float32(0.0)

    zb = z_ref[0]
    zt = zb.T
    e = e_ref[...]
    z2 = jnp.sum(zt * zt, axis=1, keepdims=True)
    e2 = jnp.sum(e * e, axis=1)
    prod = jax.lax.dot_general(
        zt, e, (((1,), (1,)), ((), ())),
        preferred_element_type=jnp.float32)
    d = z2 + e2[None, :] - 2.0 * prod

    dmin = jnp.min(d, axis=1, keepdims=True)
    iot = jax.lax.broadcasted_iota(jnp.int32, d.shape, 1)
    midx = jnp.min(jnp.where(d == dmin, iot, _N_E), axis=1,
                   keepdims=True)
    midx_row = midx.reshape(1, _PB)
    iot0 = jax.lax.broadcasted_iota(jnp.int32, (_N_E, _PB), 0)
    onehot_t = (iot0 == midx_row).astype(jnp.bfloat16)
    dims = (((1,), (0,)), ((), ()))
    zq_t = (
        jax.lax.dot_general(eth_ref[...], onehot_t, dims,
                            preferred_element_type=jnp.float32)
        + jax.lax.dot_general(etm_ref[...], onehot_t, dims,
                              preferred_element_type=jnp.float32)
    )
    zq_ref[0] = zq_t
    idx_ref[...] = midx_row.reshape(1, 1, _PB)

    sse_ref[0, 0] += jnp.sum(dmin)
    counts_ref[...] += jax.lax.dot_general(
        onehot_t, ones_ref[...], (((1,), (0,)), ((), ())),
        preferred_element_type=jnp.float32)

    @pl.when(j == _GRID - 1)
    def _final():
        n = jnp.float32(_ROWS)
        counts = counts_ref[:, 0:1]
        avg = counts / n
        loss_ref[...] = jnp.full(
            (1, 1), (1.0 + _BETA) * sse_ref[0, 0] / (n * _E_DIM), jnp.float32)
        ent = jnp.sum(avg * jnp.log(avg + 1e-10), axis=0, keepdims=True)
        perp_ref[...] = jnp.exp(-ent)
        use_ref[...] = jnp.sum((counts > 0.0).astype(jnp.int32), axis=0,
                               keepdims=True)


def kernel(z, embedding_weight):
    zv = z.reshape(_B, _E_DIM, _POS)
    et = embedding_weight.T
    et_hi = et.astype(jnp.bfloat16)
    et_mid = (et - et_hi.astype(jnp.float32)).astype(jnp.bfloat16)
    ones = jnp.ones((_PB, 128), jnp.bfloat16)
    zq_v, idxs, loss, perp, use = pl.pallas_call(
        _vq_kernel,
        grid=(_GRID,),
        in_specs=[
            pl.BlockSpec((1, _E_DIM, _PB), lambda j: (j // _PPB, 0, j % _PPB)),
            pl.BlockSpec((_N_E, _E_DIM), lambda j: (0, 0)),
            pl.BlockSpec((_E_DIM, _N_E), lambda j: (0, 0)),
            pl.BlockSpec((_E_DIM, _N_E), lambda j: (0, 0)),
            pl.BlockSpec((_PB, 128), lambda j: (0, 0)),
        ],
        out_specs=[
            pl.BlockSpec((1, _E_DIM, _PB), lambda j: (j // _PPB, 0, j % _PPB)),
            pl.BlockSpec((1, 1, _PB), lambda j: (j, 0, 0)),
            pl.BlockSpec((1, 1), lambda j: (0, 0)),
            pl.BlockSpec((1, 1), lambda j: (0, 0)),
            pl.BlockSpec((1, 1), lambda j: (0, 0)),
        ],
        out_shape=[
            jax.ShapeDtypeStruct((_B, _E_DIM, _POS), jnp.float32),
            jax.ShapeDtypeStruct((_GRID, 1, _PB), jnp.int32),
            jax.ShapeDtypeStruct((1, 1), jnp.float32),
            jax.ShapeDtypeStruct((1, 1), jnp.float32),
            jax.ShapeDtypeStruct((1, 1), jnp.int32),
        ],
        scratch_shapes=[
            pltpu.VMEM((_N_E, 128), jnp.float32),
            pltpu.SMEM((1, 1), jnp.float32),
        ],
    )(zv, embedding_weight, et_hi, et_mid, ones)
    z_q_out = zq_v.reshape(_B, _E_DIM, 32, 32)
    return (z_q_out, loss[0, 0], perp[0, 0], use[0, 0],
            idxs.reshape(_ROWS))


# speedup vs baseline: 1.7640x; 1.0003x over previous
import jax
import jax.numpy as jnp
from jax.experimental import pallas as pl
from jax.experimental.pallas import tpu as pltpu

_N_E = 1024
_E_DIM = 256
_BETA = 0.25
_B = 16
_POS = 1024
_ROWS = _B * _POS
_PB = 1024
_PPB = _POS // _PB
_GRID = _ROWS // _PB


def _vq_kernel(z_ref, e_ref, eth_ref, etm_ref, ones_ref, zq_ref, idx_ref,
               loss_ref, perp_ref, use_ref, counts_ref, sse_ref):
    j = pl.program_id(0)

    @pl.when(j == 0)
    def _init():
        counts_ref[...] = jnp.zeros_like(counts_ref)
        sse_ref[0, 0] = jnp.float32(0.0)

    zb = z_ref[0]
    zt = zb.T
    e = e_ref[...]
    z2 = jnp.sum(zt * zt, axis=1, keepdims=True)
    e2 = jnp.sum(e * e, axis=1)
    prod = jax.lax.dot_general(
        zt, e, (((1,), (1,)), ((), ())),
        preferred_element_type=jnp.float32)
    d = z2 + e2[None, :] - 2.0 * prod

    dmin = jnp.min(d, axis=1, keepdims=True)
    iot = jax.lax.broadcasted_iota(jnp.int32, d.shape, 1)
    midx = jnp.min(jnp.where(d == dmin, iot, _N_E), axis=1,
                   keepdims=True)
    midx_row = midx.reshape(1, _PB)
    iot0 = jax.lax.broadcasted_iota(jnp.int32, (_N_E, _PB), 0)
    onehot_t = (iot0 == midx_row).astype(jnp.bfloat16)
    dims = (((1,), (0,)), ((), ()))
    zq_t = (
        jax.lax.dot_general(eth_ref[...], onehot_t, dims,
                            preferred_element_type=jnp.float32)
        + jax.lax.dot_general(etm_ref[...], onehot_t, dims,
                              preferred_element_type=jnp.float32)
    )
    zq_ref[0] = zq_t
    idx_ref[...] = midx_row.reshape(1, 1, _PB)

    sse_ref[0, 0] += jnp.sum(dmin)
    counts_ref[...] += jax.lax.dot_general(
        onehot_t, ones_ref[...], (((1,), (0,)), ((), ())),
        preferred_element_type=jnp.float32)

    @pl.when(j == _GRID - 1)
    def _final():
        n = jnp.float32(_ROWS)
        counts = counts_ref[:, 0:1]
        avg = counts / n
        loss_ref[...] = jnp.full(
            (1, 1), (1.0 + _BETA) * sse_ref[0, 0] / (n * _E_DIM), jnp.float32)
        ent = jnp.sum(avg * jnp.log(avg + 1e-10), axis=0, keepdims=True)
        perp_ref[...] = jnp.exp(-ent)
        use_ref[...] = jnp.sum((counts > 0.0).astype(jnp.int32), axis=0,
                               keepdims=True)


def kernel(z, embedding_weight):
    zv = z.reshape(_B, _E_DIM, _POS)
    et = embedding_weight.T
    et_hi = et.astype(jnp.bfloat16)
    et_mid = (et - et_hi.astype(jnp.float32)).astype(jnp.bfloat16)
    ones = jnp.ones((_PB, 8), jnp.bfloat16)
    zq_v, idxs, loss, perp, use = pl.pallas_call(
        _vq_kernel,
        grid=(_GRID,),
        in_specs=[
            pl.BlockSpec((1, _E_DIM, _PB), lambda j: (j // _PPB, 0, j % _PPB)),
            pl.BlockSpec((_N_E, _E_DIM), lambda j: (0, 0)),
            pl.BlockSpec((_E_DIM, _N_E), lambda j: (0, 0)),
            pl.BlockSpec((_E_DIM, _N_E), lambda j: (0, 0)),
            pl.BlockSpec((_PB, 8), lambda j: (0, 0)),
        ],
        out_specs=[
            pl.BlockSpec((1, _E_DIM, _PB), lambda j: (j // _PPB, 0, j % _PPB)),
            pl.BlockSpec((1, 1, _PB), lambda j: (j, 0, 0)),
            pl.BlockSpec((1, 1), lambda j: (0, 0)),
            pl.BlockSpec((1, 1), lambda j: (0, 0)),
            pl.BlockSpec((1, 1), lambda j: (0, 0)),
        ],
        out_shape=[
            jax.ShapeDtypeStruct((_B, _E_DIM, _POS), jnp.float32),
            jax.ShapeDtypeStruct((_GRID, 1, _PB), jnp.int32),
            jax.ShapeDtypeStruct((1, 1), jnp.float32),
            jax.ShapeDtypeStruct((1, 1), jnp.float32),
            jax.ShapeDtypeStruct((1, 1), jnp.int32),
        ],
        scratch_shapes=[
            pltpu.VMEM((_N_E, 8), jnp.float32),
            pltpu.SMEM((1, 1), jnp.float32),
        ],
    )(zv, embedding_weight, et_hi, et_mid, ones)
    z_q_out = zq_v.reshape(_B, _E_DIM, 32, 32)
    return (z_q_out, loss[0, 0], perp[0, 0], use[0, 0],
            idxs.reshape(_ROWS))
